# baseline (speedup 1.0000x reference)
import jax
import jax.numpy as jnp
from jax.experimental import pallas as pl
from jax.experimental.pallas import tpu as pltpu

NUM_Q = 4
K = 8192
D = 32
B = 16
N = 1024
BETA = 0.25

M = B * N
MT = 512
CW = 4096
NC = K // CW


def _sum32_lanes(v):
    s = ((v[..., 0:8] + v[..., 8:16]) + v[..., 16:24]) + v[..., 24:32]
    s = s[..., 0:4] + s[..., 4:8]
    s = s[..., 0:2] + s[..., 2:4]
    return s[..., 0:1] + s[..., 1:2]


def _sum32_rows(v):
    s = ((v[0:8, :] + v[8:16, :]) + v[16:24, :]) + v[24:32, :]
    s = s[0:4, :] + s[4:8, :]
    s = s[0:2, :] + s[2:4, :]
    return s[0:1, :] + s[1:2, :]


def _body(x_ref, cbt_ref, wt_ref, b_ref,
          qout_ref, idx_ref, loss_ref,
          hi_ref, mid_ref, lo_ref, c2_ref):
    i = pl.program_id(0)

    @pl.when(i == 0)
    def _init():
        for q in range(NUM_Q):
            impt = jnp.dot(wt_ref[q].astype(jnp.bfloat16),
                           cbt_ref[q].astype(jnp.bfloat16),
                           preferred_element_type=jnp.float32) + b_ref[q]
            hi = impt.astype(jnp.bfloat16)
            r1 = impt - hi.astype(jnp.float32)
            mid = r1.astype(jnp.bfloat16)
            lo = (r1 - mid.astype(jnp.float32)).astype(jnp.bfloat16)
            hi_ref[q] = hi
            mid_ref[q] = mid
            lo_ref[q] = lo
            c2_ref[q] = _sum32_rows(impt * impt)
        loss_ref[...] = jnp.zeros((1, 1), jnp.float32)

    residual = x_ref[...]
    acc = jnp.zeros_like(residual)
    lsum = jnp.zeros((1, 1), dtype=jnp.float32)
    for q in range(NUM_Q):
        resb = residual.astype(jnp.bfloat16)
        r2 = _sum32_lanes(residual * residual)
        run_v = jnp.full((MT, 1), jnp.inf, jnp.float32)
        run_i = jnp.zeros((MT, 1), jnp.int32)
        for c in range(NC):
            ein = jnp.dot(resb, hi_ref[q, :, c * CW:(c + 1) * CW],
                          preferred_element_type=jnp.float32)
            d2 = (r2 - 2.0 * ein) + c2_ref[q, :, c * CW:(c + 1) * CW]
            m = jnp.min(d2, axis=-1, keepdims=True)
            li = (jnp.argmin(d2, axis=-1).astype(jnp.int32)[:, None]
                  + c * CW)
            better = m < run_v
            run_i = jnp.where(better, li, run_i)
            run_v = jnp.where(better, m, run_v)
            if q > 0:
                run_v = run_v.astype(jnp.bfloat16).astype(jnp.float32)
        idx = run_i
        ohb = (jax.lax.broadcasted_iota(jnp.int32, (MT, K), 1)
               == idx).astype(jnp.bfloat16)
        dn = (((1,), (1,)), ((), ()))
        quant = ((jax.lax.dot_general(ohb, hi_ref[q], dn,
                                      preferred_element_type=jnp.float32)
                  + jax.lax.dot_general(ohb, mid_ref[q], dn,
                                        preferred_element_type=jnp.float32))
                 + jax.lax.dot_general(ohb, lo_ref[q], dn,
                                       preferred_element_type=jnp.float32))
        diff = quant - residual
        lsum = lsum + jnp.sum(diff * diff).reshape(1, 1)
        acc = acc + (residual + (quant - residual))
        idx_ref[q, :] = idx[:, 0]
        residual = residual - quant
    qout_ref[...] = acc
    loss_ref[...] += lsum


def kernel(x, codebooks, W, b):
    x2 = x.reshape(M, D)
    cbt = jnp.swapaxes(codebooks, 1, 2)
    wt = jnp.swapaxes(W, 1, 2)
    b3 = b.reshape(NUM_Q, D, 1)
    grid = (M // MT,)
    qout, idxs, lsum = pl.pallas_call(
        _body,
        grid=grid,
        in_specs=[
            pl.BlockSpec((MT, D), lambda i: (i, 0)),
            pl.BlockSpec((NUM_Q, D, K), lambda i: (0, 0, 0)),
            pl.BlockSpec((NUM_Q, D, D), lambda i: (0, 0, 0)),
            pl.BlockSpec((NUM_Q, D, 1), lambda i: (0, 0, 0)),
        ],
        out_specs=[
            pl.BlockSpec((MT, D), lambda i: (i, 0)),
            pl.BlockSpec((NUM_Q, MT), lambda i: (0, i)),
            pl.BlockSpec((1, 1), lambda i: (0, 0)),
        ],
        out_shape=[
            jax.ShapeDtypeStruct((M, D), jnp.float32),
            jax.ShapeDtypeStruct((NUM_Q, M), jnp.int32),
            jax.ShapeDtypeStruct((1, 1), jnp.float32),
        ],
        scratch_shapes=[
            pltpu.VMEM((NUM_Q, D, K), jnp.bfloat16),
            pltpu.VMEM((NUM_Q, D, K), jnp.bfloat16),
            pltpu.VMEM((NUM_Q, D, K), jnp.bfloat16),
            pltpu.VMEM((NUM_Q, 1, K), jnp.float32),
        ],
    )(x2, cbt, wt, b3)
    quantized_out = qout.reshape(B, N, D)
    indices = idxs.reshape(NUM_Q, B, N)
    total_loss = (1.0 + BETA) * lsum[0, 0] / jnp.float32(M * D)
    return quantized_out, indices, total_loss


# speedup vs baseline: 1.1463x; 1.1463x over previous
import jax
import jax.numpy as jnp
from jax.experimental import pallas as pl
from jax.experimental.pallas import tpu as pltpu

NUM_Q = 4
K = 8192
D = 32
B = 16
N = 1024
BETA = 0.25

M = B * N
MT = 256
CW = 4096
NC = K // CW


def _sum32_lanes(v):
    s = ((v[..., 0:8] + v[..., 8:16]) + v[..., 16:24]) + v[..., 24:32]
    s = s[..., 0:4] + s[..., 4:8]
    s = s[..., 0:2] + s[..., 2:4]
    return s[..., 0:1] + s[..., 1:2]


def _sum32_rows(v):
    s = ((v[0:8, :] + v[8:16, :]) + v[16:24, :]) + v[24:32, :]
    s = s[0:4, :] + s[4:8, :]
    s = s[0:2, :] + s[2:4, :]
    return s[0:1, :] + s[1:2, :]


def _body(x_ref, cbt_ref, wt_ref, b_ref,
          qout_ref, idx_ref, loss_ref,
          hi_ref, mid_ref, lo_ref, c2_ref):
    i = pl.program_id(0)

    @pl.when(i == 0)
    def _init():
        for q in range(NUM_Q):
            impt = jnp.dot(wt_ref[q].astype(jnp.bfloat16),
                           cbt_ref[q].astype(jnp.bfloat16),
                           preferred_element_type=jnp.float32) + b_ref[q]
            hi = impt.astype(jnp.bfloat16)
            r1 = impt - hi.astype(jnp.float32)
            mid = r1.astype(jnp.bfloat16)
            lo = (r1 - mid.astype(jnp.float32)).astype(jnp.bfloat16)
            hi_ref[q] = hi
            mid_ref[q] = mid
            lo_ref[q] = lo
            c2_ref[q] = _sum32_rows(impt * impt)
        loss_ref[...] = jnp.zeros((1, 1), jnp.float32)

    residual = x_ref[...]
    acc = jnp.zeros_like(residual)
    lsum = jnp.zeros((1, 1), dtype=jnp.float32)
    for q in range(NUM_Q):
        resb = residual.astype(jnp.bfloat16)
        r2 = _sum32_lanes(residual * residual)
        run_v = jnp.full((MT, 1), jnp.inf, jnp.float32)
        run_i = jnp.zeros((MT, 1), jnp.int32)
        for c in range(NC):
            ein = jnp.dot(resb, hi_ref[q, :, c * CW:(c + 1) * CW],
                          preferred_element_type=jnp.float32)
            d2 = (r2 - 2.0 * ein) + c2_ref[q, :, c * CW:(c + 1) * CW]
            m = jnp.min(d2, axis=-1, keepdims=True)
            li = (jnp.argmin(d2, axis=-1).astype(jnp.int32)[:, None]
                  + c * CW)
            better = m < run_v
            run_i = jnp.where(better, li, run_i)
            run_v = jnp.where(better, m, run_v)
            if q > 0:
                run_v = run_v.astype(jnp.bfloat16).astype(jnp.float32)
        idx = run_i
        ohb = (jax.lax.broadcasted_iota(jnp.int32, (MT, K), 1)
               == idx).astype(jnp.bfloat16)
        dn = (((1,), (1,)), ((), ()))
        quant = ((jax.lax.dot_general(ohb, hi_ref[q], dn,
                                      preferred_element_type=jnp.float32)
                  + jax.lax.dot_general(ohb, mid_ref[q], dn,
                                        preferred_element_type=jnp.float32))
                 + jax.lax.dot_general(ohb, lo_ref[q], dn,
                                       preferred_element_type=jnp.float32))
        diff = quant - residual
        lsum = lsum + jnp.sum(diff * diff).reshape(1, 1)
        acc = acc + (residual + (quant - residual))
        idx_ref[q, :] = idx[:, 0]
        residual = residual - quant
    qout_ref[...] = acc
    loss_ref[...] += lsum


def kernel(x, codebooks, W, b):
    x2 = x.reshape(M, D)
    cbt = jnp.swapaxes(codebooks, 1, 2)
    wt = jnp.swapaxes(W, 1, 2)
    b3 = b.reshape(NUM_Q, D, 1)
    grid = (M // MT,)
    qout, idxs, lsum = pl.pallas_call(
        _body,
        grid=grid,
        in_specs=[
            pl.BlockSpec((MT, D), lambda i: (i, 0)),
            pl.BlockSpec((NUM_Q, D, K), lambda i: (0, 0, 0)),
            pl.BlockSpec((NUM_Q, D, D), lambda i: (0, 0, 0)),
            pl.BlockSpec((NUM_Q, D, 1), lambda i: (0, 0, 0)),
        ],
        out_specs=[
            pl.BlockSpec((MT, D), lambda i: (i, 0)),
            pl.BlockSpec((NUM_Q, MT), lambda i: (0, i)),
            pl.BlockSpec((1, 1), lambda i: (0, 0)),
        ],
        out_shape=[
            jax.ShapeDtypeStruct((M, D), jnp.float32),
            jax.ShapeDtypeStruct((NUM_Q, M), jnp.int32),
            jax.ShapeDtypeStruct((1, 1), jnp.float32),
        ],
        scratch_shapes=[
            pltpu.VMEM((NUM_Q, D, K), jnp.bfloat16),
            pltpu.VMEM((NUM_Q, D, K), jnp.bfloat16),
            pltpu.VMEM((NUM_Q, D, K), jnp.bfloat16),
            pltpu.VMEM((NUM_Q, 1, K), jnp.float32),
        ],
    )(x2, cbt, wt, b3)
    quantized_out = qout.reshape(B, N, D)
    indices = idxs.reshape(NUM_Q, B, N)
    total_loss = (1.0 + BETA) * lsum[0, 0] / jnp.float32(M * D)
    return quantized_out, indices, total_loss


# speedup vs baseline: 1.1566x; 1.0090x over previous
import jax
import jax.numpy as jnp
from jax.experimental import pallas as pl
from jax.experimental.pallas import tpu as pltpu

NUM_Q = 4
K = 8192
D = 32
B = 16
N = 1024
BETA = 0.25

M = B * N
MT = 256
CW = 4096
NC = K // CW


def _sum32_lanes(v):
    s = ((v[..., 0:8] + v[..., 8:16]) + v[..., 16:24]) + v[..., 24:32]
    s = s[..., 0:4] + s[..., 4:8]
    s = s[..., 0:2] + s[..., 2:4]
    return s[..., 0:1] + s[..., 1:2]


def _sum32_rows(v):
    s = ((v[0:8, :] + v[8:16, :]) + v[16:24, :]) + v[24:32, :]
    s = s[0:4, :] + s[4:8, :]
    s = s[0:2, :] + s[2:4, :]
    return s[0:1, :] + s[1:2, :]


def _body(x_ref, cbt_ref, wt_ref, b_ref,
          qout_ref, idx_ref, loss_ref,
          hi_ref, mid_ref, lo_ref, c2_ref):
    i = pl.program_id(0)

    @pl.when(i == 0)
    def _init():
        for q in range(NUM_Q):
            impt = jnp.dot(wt_ref[q].astype(jnp.bfloat16),
                           cbt_ref[q].astype(jnp.bfloat16),
                           preferred_element_type=jnp.float32) + b_ref[q]
            hi = impt.astype(jnp.bfloat16)
            r1 = impt - hi.astype(jnp.float32)
            mid = r1.astype(jnp.bfloat16)
            lo = (r1 - mid.astype(jnp.float32)).astype(jnp.bfloat16)
            hi_ref[q] = hi
            mid_ref[q] = mid
            lo_ref[q] = lo
            c2_ref[q] = _sum32_rows(impt * impt)
        loss_ref[...] = jnp.zeros((1, 1), jnp.float32)

    residual = x_ref[...]
    acc = jnp.zeros_like(residual)
    lsum = jnp.zeros((1, 1), dtype=jnp.float32)
    for q in range(NUM_Q):
        resb = residual.astype(jnp.bfloat16)
        resb2 = resb * jnp.bfloat16(-2.0)
        r2 = _sum32_lanes(residual * residual)
        run_v = jnp.full((MT, 1), jnp.inf, jnp.float32)
        run_i = jnp.zeros((MT, 1), jnp.int32)
        for c in range(NC):
            ein2 = jnp.dot(resb2, hi_ref[q, :, c * CW:(c + 1) * CW],
                           preferred_element_type=jnp.float32)
            d2 = (r2 + ein2) + c2_ref[q, :, c * CW:(c + 1) * CW]
            m = jnp.min(d2, axis=-1, keepdims=True)
            li = (jnp.argmin(d2, axis=-1).astype(jnp.int32)[:, None]
                  + c * CW)
            better = m < run_v
            run_i = jnp.where(better, li, run_i)
            run_v = jnp.where(better, m, run_v)
            if q > 0:
                run_v = run_v.astype(jnp.bfloat16).astype(jnp.float32)
        idx = run_i
        ohb = (jax.lax.broadcasted_iota(jnp.int32, (MT, K), 1)
               == idx).astype(jnp.bfloat16)
        dn = (((1,), (1,)), ((), ()))
        quant = ((jax.lax.dot_general(ohb, hi_ref[q], dn,
                                      preferred_element_type=jnp.float32)
                  + jax.lax.dot_general(ohb, mid_ref[q], dn,
                                        preferred_element_type=jnp.float32))
                 + jax.lax.dot_general(ohb, lo_ref[q], dn,
                                       preferred_element_type=jnp.float32))
        diff = quant - residual
        lsum = lsum + jnp.sum(diff * diff).reshape(1, 1)
        acc = acc + (residual + (quant - residual))
        idx_ref[q, :] = idx[:, 0]
        residual = residual - quant
    qout_ref[...] = acc
    loss_ref[...] += lsum


def kernel(x, codebooks, W, b):
    x2 = x.reshape(M, D)
    cbt = jnp.swapaxes(codebooks, 1, 2)
    wt = jnp.swapaxes(W, 1, 2)
    b3 = b.reshape(NUM_Q, D, 1)
    grid = (M // MT,)
    qout, idxs, lsum = pl.pallas_call(
        _body,
        grid=grid,
        in_specs=[
            pl.BlockSpec((MT, D), lambda i: (i, 0)),
            pl.BlockSpec((NUM_Q, D, K), lambda i: (0, 0, 0)),
            pl.BlockSpec((NUM_Q, D, D), lambda i: (0, 0, 0)),
            pl.BlockSpec((NUM_Q, D, 1), lambda i: (0, 0, 0)),
        ],
        out_specs=[
            pl.BlockSpec((MT, D), lambda i: (i, 0)),
            pl.BlockSpec((NUM_Q, MT), lambda i: (0, i)),
            pl.BlockSpec((1, 1), lambda i: (0, 0)),
        ],
        out_shape=[
            jax.ShapeDtypeStruct((M, D), jnp.float32),
            jax.ShapeDtypeStruct((NUM_Q, M), jnp.int32),
            jax.ShapeDtypeStruct((1, 1), jnp.float32),
        ],
        scratch_shapes=[
            pltpu.VMEM((NUM_Q, D, K), jnp.bfloat16),
            pltpu.VMEM((NUM_Q, D, K), jnp.bfloat16),
            pltpu.VMEM((NUM_Q, D, K), jnp.bfloat16),
            pltpu.VMEM((NUM_Q, 1, K), jnp.float32),
        ],
    )(x2, cbt, wt, b3)
    quantized_out = qout.reshape(B, N, D)
    indices = idxs.reshape(NUM_Q, B, N)
    total_loss = (1.0 + BETA) * lsum[0, 0] / jnp.float32(M * D)
    return quantized_out, indices, total_loss


# speedup vs baseline: 1.7338x; 1.4990x over previous
import jax
import jax.numpy as jnp
from jax.experimental import pallas as pl
from jax.experimental.pallas import tpu as pltpu

NUM_Q = 4
K = 8192
D = 32
B = 16
N = 1024
BETA = 0.25

M = B * N
MT = 256
CW = 4096
NC = K // CW


def _sum32_lanes(v):
    s = ((v[..., 0:8] + v[..., 8:16]) + v[..., 16:24]) + v[..., 24:32]
    s = s[..., 0:4] + s[..., 4:8]
    s = s[..., 0:2] + s[..., 2:4]
    return s[..., 0:1] + s[..., 1:2]


def _sum32_rows(v):
    s = ((v[0:8, :] + v[8:16, :]) + v[16:24, :]) + v[24:32, :]
    s = s[0:4, :] + s[4:8, :]
    s = s[0:2, :] + s[2:4, :]
    return s[0:1, :] + s[1:2, :]


def _body(x_ref, cbt_ref, wt_ref, b_ref,
          qout_ref, idx_ref, loss_ref,
          hi_ref, hikd_ref, midkd_ref, lokd_ref, c2_ref):
    i = pl.program_id(0)

    @pl.when(i == 0)
    def _init():
        for q in range(NUM_Q):
            impt = jnp.dot(wt_ref[q].astype(jnp.bfloat16),
                           cbt_ref[q].astype(jnp.bfloat16),
                           preferred_element_type=jnp.float32) + b_ref[q]
            hi = impt.astype(jnp.bfloat16)
            r1 = impt - hi.astype(jnp.float32)
            mid = r1.astype(jnp.bfloat16)
            lo = (r1 - mid.astype(jnp.float32)).astype(jnp.bfloat16)
            hi_ref[q] = hi
            hikd_ref[q] = hi.T
            midkd_ref[q] = mid.T
            lokd_ref[q] = lo.T
            c2_ref[q] = _sum32_rows(impt * impt)
        loss_ref[...] = jnp.zeros((1, 1), jnp.float32)

    residual = x_ref[...]
    acc = jnp.zeros_like(residual)
    lsum = jnp.zeros((1, 1), dtype=jnp.float32)
    for q in range(NUM_Q):
        resb = residual.astype(jnp.bfloat16)
        resb2 = resb * jnp.bfloat16(-2.0)
        r2 = _sum32_lanes(residual * residual)
        run_v = jnp.full((MT, 1), jnp.inf, jnp.float32)
        run_i = jnp.zeros((MT, 1), jnp.int32)
        for c in range(NC):
            ein2 = jnp.dot(resb2, hi_ref[q, :, c * CW:(c + 1) * CW],
                           preferred_element_type=jnp.float32)
            d2 = (r2 + ein2) + c2_ref[q, :, c * CW:(c + 1) * CW]
            m = jnp.min(d2, axis=-1, keepdims=True)
            li = (jnp.argmin(d2, axis=-1).astype(jnp.int32)[:, None]
                  + c * CW)
            better = m < run_v
            run_i = jnp.where(better, li, run_i)
            run_v = jnp.where(better, m, run_v)
            if q > 0:
                run_v = run_v.astype(jnp.bfloat16).astype(jnp.float32)
        idx = run_i
        ohb = (jax.lax.broadcasted_iota(jnp.int32, (MT, K), 1)
               == idx).astype(jnp.bfloat16)
        quant = ((jnp.dot(ohb, hikd_ref[q],
                          preferred_element_type=jnp.float32)
                  + jnp.dot(ohb, midkd_ref[q],
                            preferred_element_type=jnp.float32))
                 + jnp.dot(ohb, lokd_ref[q],
                           preferred_element_type=jnp.float32))
        diff = quant - residual
        lsum = lsum + jnp.sum(diff * diff).reshape(1, 1)
        acc = acc + (residual + (quant - residual))
        idx_ref[q, :] = idx[:, 0]
        residual = residual - quant
    qout_ref[...] = acc
    loss_ref[...] += lsum


def kernel(x, codebooks, W, b):
    x2 = x.reshape(M, D)
    cbt = jnp.swapaxes(codebooks, 1, 2)
    wt = jnp.swapaxes(W, 1, 2)
    b3 = b.reshape(NUM_Q, D, 1)
    grid = (M // MT,)
    qout, idxs, lsum = pl.pallas_call(
        _body,
        grid=grid,
        in_specs=[
            pl.BlockSpec((MT, D), lambda i: (i, 0)),
            pl.BlockSpec((NUM_Q, D, K), lambda i: (0, 0, 0)),
            pl.BlockSpec((NUM_Q, D, D), lambda i: (0, 0, 0)),
            pl.BlockSpec((NUM_Q, D, 1), lambda i: (0, 0, 0)),
        ],
        out_specs=[
            pl.BlockSpec((MT, D), lambda i: (i, 0)),
            pl.BlockSpec((NUM_Q, MT), lambda i: (0, i)),
            pl.BlockSpec((1, 1), lambda i: (0, 0)),
        ],
        out_shape=[
            jax.ShapeDtypeStruct((M, D), jnp.float32),
            jax.ShapeDtypeStruct((NUM_Q, M), jnp.int32),
            jax.ShapeDtypeStruct((1, 1), jnp.float32),
        ],
        scratch_shapes=[
            pltpu.VMEM((NUM_Q, D, K), jnp.bfloat16),
            pltpu.VMEM((NUM_Q, K, D), jnp.bfloat16),
            pltpu.VMEM((NUM_Q, K, D), jnp.bfloat16),
            pltpu.VMEM((NUM_Q, K, D), jnp.bfloat16),
            pltpu.VMEM((NUM_Q, 1, K), jnp.float32),
        ],
    )(x2, cbt, wt, b3)
    quantized_out = qout.reshape(B, N, D)
    indices = idxs.reshape(NUM_Q, B, N)
    total_loss = (1.0 + BETA) * lsum[0, 0] / jnp.float32(M * D)
    return quantized_out, indices, total_loss
